# baseline (speedup 1.0000x reference)
import functools

import jax
import jax.numpy as jnp
from jax import lax
from jax.experimental import pallas as pl
from jax.experimental.pallas import tpu as pltpu
from jax.experimental.pallas import tpu_sc as plsc

BATCH = 16384
D = 128
NC = 2
NS = 16
NW = NC * NS
BPW = BATCH // NW
CH = 128
NCH = BPW // CH
SIZES = (32, 96, 128, 128, 128)
OFFS = (0, 32, 128, 256, 384)

_mesh = plsc.VectorSubcoreMesh(core_axis_name="c", subcore_axis_name="s")


def _idx_slice(idx_v, k):
    if k == 0:
        return idx_v.at[0, pl.ds(0, 32)]
    if k == 1:
        return idx_v.at[0, pl.ds(32, 96)]
    return idx_v.at[k - 1]


@functools.partial(
    pl.kernel,
    mesh=_mesh,
    out_type=(
        jax.ShapeDtypeStruct((BATCH, D), jnp.float32),
        jax.ShapeDtypeStruct((BATCH, D), jnp.float32),
    ),
    scratch_types=[
        pltpu.VMEM((NCH, CH), jnp.int32),
        pltpu.VMEM((NCH, CH), jnp.int32),
        pltpu.VMEM((768, D), jnp.float32),
        pltpu.SemaphoreType.DMA((2,)),
        pltpu.SemaphoreType.DMA((8,)),
        pltpu.SemaphoreType.DMA((8,)),
    ],
)
def _sc_gather(uids, iids, utab, itab, uout, iout, uidx_v, iidx_v,
               bufs, isem, gsem, wsem):
    wid = lax.axis_index("s") * NC + lax.axis_index("c")
    base = wid * BPW

    ldu = pltpu.async_copy(uids.at[wid], uidx_v, isem.at[0])
    ldi = pltpu.async_copy(iids.at[wid], iidx_v, isem.at[1])
    ldu.wait()
    ldi.wait()

    UB, IB = 0, 384

    def chunk(c):
        k = c // 2
        if c % 2 == 0:
            return utab, _idx_slice(uidx_v, k), uout, UB, k
        return itab, _idx_slice(iidx_v, k), iout, IB, k

    def buf_slice(bb, k):
        boff = bb + (OFFS[k] if k < 4 else 0)
        return bufs.at[pl.ds(boff, SIZES[k])]

    def fire(c):
        tab, idx, _, bb, k = chunk(c)
        return pltpu.async_copy(tab.at[idx], buf_slice(bb, k),
                                gsem.at[c % 8])

    def writeback(c):
        _, _, out, bb, k = chunk(c)
        return pltpu.async_copy(buf_slice(bb, k),
                                out.at[pl.ds(base + OFFS[k], SIZES[k])],
                                wsem.at[c % 8])

    g = [fire(c) for c in range(8)]
    wb = [None] * 10

    for c in range(4):
        g[c].wait()
        wb[c] = writeback(c)
    wb[0].wait()
    wb[2].wait()
    g.append(fire(8))
    wb[1].wait()
    wb[3].wait()
    g.append(fire(9))
    for c in range(4, 10):
        g[c].wait()
        wb[c] = writeback(c)
    for c in range(4, 10):
        wb[c].wait()


def kernel(user_ids, item_ids, user_table, item_table):
    uids = user_ids.astype(jnp.int32).reshape(NW, NCH, CH)
    iids = item_ids.astype(jnp.int32).reshape(NW, NCH, CH)
    return _sc_gather(uids, iids, user_table, item_table)


# speedup vs baseline: 1.0309x; 1.0309x over previous
import functools

import jax
import jax.numpy as jnp
from jax import lax
from jax.experimental import pallas as pl
from jax.experimental.pallas import tpu as pltpu
from jax.experimental.pallas import tpu_sc as plsc

BATCH = 16384
D = 128
NC = 2
NS = 16
NW = NC * NS
BPW = BATCH // NW
CH = 128
NCH = BPW // CH
NCHUNKS = 2 * NCH
NBUF = 7

_mesh = plsc.VectorSubcoreMesh(core_axis_name="c", subcore_axis_name="s")


@functools.partial(
    pl.kernel,
    mesh=_mesh,
    out_type=(
        jax.ShapeDtypeStruct((BATCH, D), jnp.float32),
        jax.ShapeDtypeStruct((BATCH, D), jnp.float32),
    ),
    scratch_types=[
        pltpu.VMEM((NCH, CH), jnp.int32),
        pltpu.VMEM((NCH, CH), jnp.int32),
        pltpu.VMEM((NBUF, CH, D), jnp.float32),
        pltpu.SemaphoreType.DMA((2,)),
        pltpu.SemaphoreType.DMA((NCHUNKS,)),
        pltpu.SemaphoreType.DMA((NCHUNKS,)),
    ],
)
def _sc_gather(uids, iids, utab, itab, uout, iout, uidx_v, iidx_v,
               bufs, isem, gsem, wsem):
    wid = lax.axis_index("s") * NC + lax.axis_index("c")
    base = wid * BPW

    ldu = pltpu.async_copy(uids.at[wid], uidx_v, isem.at[0])
    ldi = pltpu.async_copy(iids.at[wid], iidx_v, isem.at[1])
    ldu.wait()
    ldi.wait()

    def chunk(c):
        j = c // 2
        if c % 2 == 0:
            return utab, uidx_v.at[j], uout, base + j * CH
        return itab, iidx_v.at[j], iout, base + j * CH

    def fire(c):
        tab, idx, _, _ = chunk(c)
        return pltpu.async_copy(tab.at[idx], bufs.at[c % NBUF], gsem.at[c])

    DEPTH = NBUF
    gathers = [fire(c) for c in range(DEPTH)]
    writebacks = []
    for c in range(NCHUNKS):
        _, _, out, off = chunk(c)
        gathers[c].wait()
        writebacks.append(pltpu.async_copy(bufs.at[c % NBUF],
                                           out.at[pl.ds(off, CH)], wsem.at[c]))
        nxt = c + DEPTH
        if nxt < NCHUNKS:
            if nxt >= NBUF:
                writebacks[nxt - NBUF].wait()
            gathers.append(fire(nxt))

    for c in range(NCHUNKS - NBUF, NCHUNKS):
        writebacks[c].wait()


def kernel(user_ids, item_ids, user_table, item_table):
    uids = user_ids.astype(jnp.int32).reshape(NW, NCH, CH)
    iids = item_ids.astype(jnp.int32).reshape(NW, NCH, CH)
    return _sc_gather(uids, iids, user_table, item_table)
